# baseline (speedup 1.0000x reference)
import functools

import jax
import jax.numpy as jnp
from jax import lax
from jax.experimental import pallas as pl
from jax.experimental.pallas import tpu as pltpu
from jax.experimental.pallas import tpu_sc as plsc


def _sc_gather(table, idx):
    n = idx.shape[0]
    d = table.shape[1]
    info = plsc.get_sparse_core_info()
    nw = info.num_cores * info.num_subcores
    n_per_w = n // nw
    mesh = plsc.VectorSubcoreMesh(core_axis_name="c", subcore_axis_name="s")

    @functools.partial(
        pl.kernel,
        mesh=mesh,
        out_type=jax.ShapeDtypeStruct((n, d), jnp.float32),
        scratch_types=[
            pltpu.VMEM((n_per_w,), jnp.int32),
            pltpu.VMEM((n_per_w, d), jnp.float32),
            pltpu.SemaphoreType.DMA,
        ],
    )
    def gather_kernel(table_hbm, idx_hbm, out_hbm, idx_v, rows_v, sem):
        wid = lax.axis_index("s") * info.num_cores + lax.axis_index("c")
        base = wid * n_per_w
        pltpu.sync_copy(idx_hbm.at[pl.ds(base, n_per_w)], idx_v)
        pltpu.async_copy(table_hbm.at[idx_v], rows_v, sem).wait()
        pltpu.sync_copy(rows_v, out_hbm.at[pl.ds(base, n_per_w)])

    return gather_kernel(table, idx)


def _mm_body(a_ref, w_ref, b_ref, o_ref):
    acc = (
        jnp.dot(
            a_ref[...].astype(jnp.bfloat16),
            w_ref[...].astype(jnp.bfloat16),
            preferred_element_type=jnp.float32,
        )
        + b_ref[...]
    )
    o_ref[...] = acc.astype(o_ref.dtype)


def _matmul_bias(a, w_t, bias, block_m=2048, block_n=512,
                 out_dtype=jnp.float32, gate_perm=False):
    m, k = a.shape
    n = w_t.shape[1]

    if gate_perm:
        assert n // block_n == 8

        def wmap(i, j):
            return (0, 2 * (j % 4) + j // 4)
    else:
        def wmap(i, j):
            return (0, j)

    return pl.pallas_call(
        _mm_body,
        grid=(m // block_m, n // block_n),
        in_specs=[
            pl.BlockSpec((block_m, k), lambda i, j: (i, 0)),
            pl.BlockSpec((k, block_n), wmap),
            pl.BlockSpec((1, block_n), lambda i, j: (0, wmap(i, j)[1])),
        ],
        out_specs=pl.BlockSpec((block_m, block_n), lambda i, j: (i, j)),
        out_shape=jax.ShapeDtypeStruct((m, n), out_dtype),
    )(a, w_t, bias)


def _gate_src_block(j):
    return 2 * (j % 4) + j // 4


def _lstm_scan(xg, w_hh_t, unroll=4):
    s, b, g4 = xg.shape
    h_dim = w_hh_t.shape[0]
    hh = h_dim // 2
    g2 = g4 // 2
    nb = g4 // hh

    def act_half(gates, c_half):
        gi = jax.nn.sigmoid(gates[:, :hh])
        gf = jax.nn.sigmoid(gates[:, hh : 2 * hh])
        gg = jnp.tanh(gates[:, 2 * hh : 3 * hh])
        go = jax.nn.sigmoid(gates[:, 3 * hh :])
        c_new = gf * c_half + gi * gg
        h_new = go * jnp.tanh(c_new)
        return h_new, c_new

    def body(x_ref, w_ref, y_ref, h_ref, c_ref, wb_ref, ha_ref, hb_ref):
        t = pl.program_id(0)

        @pl.when(t == 0)
        def _():
            for j in range(nb):
                src = _gate_src_block(j) * hh
                wb_ref[:, j * hh : (j + 1) * hh] = (
                    w_ref[:, src : src + hh].astype(jnp.bfloat16)
                )
            h_ref[...] = jnp.zeros_like(h_ref)
            c_ref[...] = jnp.zeros_like(c_ref)
            ha_ref[...] = jnp.zeros_like(ha_ref)
            hb_ref[...] = jnp.zeros_like(hb_ref)

        h_a = ha_ref[...]
        h_b = hb_ref[...]
        c = c_ref[...]
        c_a = c[:, :hh]
        c_b = c[:, hh:]
        w = wb_ref[...]
        for u in range(unroll):
            ga = (
                x_ref[u, :, :g2]
                + jnp.dot(h_a, w[:hh, :g2], preferred_element_type=jnp.float32)
                + jnp.dot(h_b, w[hh:, :g2], preferred_element_type=jnp.float32)
            )
            gb = (
                x_ref[u, :, g2:]
                + jnp.dot(h_a, w[:hh, g2:], preferred_element_type=jnp.float32)
                + jnp.dot(h_b, w[hh:, g2:], preferred_element_type=jnp.float32)
            )
            ha_new, c_a = act_half(ga, c_a)
            h_a = ha_new.astype(jnp.bfloat16)
            hb_new, c_b = act_half(gb, c_b)
            h_b = hb_new.astype(jnp.bfloat16)
            y_ref[u, :, :hh] = h_a
            y_ref[u, :, hh:] = h_b
        h_ref[:, :hh] = ha_new
        h_ref[:, hh:] = hb_new
        c_ref[:, :hh] = c_a
        c_ref[:, hh:] = c_b
        ha_ref[...] = h_a
        hb_ref[...] = h_b

    return pl.pallas_call(
        body,
        grid=(s // unroll,),
        in_specs=[
            pl.BlockSpec((unroll, b, g4), lambda t: (t, 0, 0)),
            pl.BlockSpec((h_dim, g4), lambda t: (0, 0)),
        ],
        out_specs=[
            pl.BlockSpec((unroll, b, h_dim), lambda t: (t, 0, 0)),
            pl.BlockSpec((b, h_dim), lambda t: (0, 0)),
            pl.BlockSpec((b, h_dim), lambda t: (0, 0)),
        ],
        out_shape=[
            jax.ShapeDtypeStruct((s, b, h_dim), jnp.bfloat16),
            jax.ShapeDtypeStruct((b, h_dim), jnp.float32),
            jax.ShapeDtypeStruct((b, h_dim), jnp.float32),
        ],
        scratch_shapes=[
            pltpu.VMEM((h_dim, g4), jnp.bfloat16),
            pltpu.VMEM((b, hh), jnp.bfloat16),
            pltpu.VMEM((b, hh), jnp.bfloat16),
        ],
    )(xg, w_hh_t)


def kernel(x, emb, W_ih0, W_hh0, b_ih0, b_hh0, W_ih1, W_hh1, b_ih1, b_hh1, W_out, b_out):
    b, s = x.shape
    h_dim = W_hh0.shape[1]

    idx = x.T.reshape(-1).astype(jnp.int32)
    e = _sc_gather(emb, idx)

    xg0 = _matmul_bias(
        e, W_ih0.T, (b_ih0 + b_hh0)[None, :],
        block_m=b * s, out_dtype=jnp.bfloat16, gate_perm=True,
    )
    y0, h0, c0 = _lstm_scan(xg0.reshape(s, b, -1), W_hh0.T)

    xg1 = _matmul_bias(
        y0.reshape(s * b, h_dim), W_ih1.T, (b_ih1 + b_hh1)[None, :],
        block_m=b * s, out_dtype=jnp.bfloat16, gate_perm=True,
    )
    y1, h1, c1 = _lstm_scan(xg1.reshape(s, b, -1), W_hh1.T)

    a = jnp.transpose(y1, (1, 0, 2)).reshape(b * s, h_dim)
    out = _matmul_bias(a, W_out.T, b_out[None, :], block_m=b * s).reshape(b, s, -1)

    h_n = jnp.stack([h0, h1], axis=0)
    c_n = jnp.stack([c0, c1], axis=0)
    return (out, h_n, c_n)


# speedup vs baseline: 1.0128x; 1.0128x over previous
import functools

import jax
import jax.numpy as jnp
from jax import lax
from jax.experimental import pallas as pl
from jax.experimental.pallas import tpu as pltpu
from jax.experimental.pallas import tpu_sc as plsc


def _sc_gather(table, idx):
    n = idx.shape[0]
    d = table.shape[1]
    info = plsc.get_sparse_core_info()
    nw = info.num_cores * info.num_subcores
    n_per_w = n // nw
    mesh = plsc.VectorSubcoreMesh(core_axis_name="c", subcore_axis_name="s")

    @functools.partial(
        pl.kernel,
        mesh=mesh,
        out_type=jax.ShapeDtypeStruct((n, d), jnp.float32),
        scratch_types=[
            pltpu.VMEM((n_per_w,), jnp.int32),
            pltpu.VMEM((n_per_w, d), jnp.float32),
            pltpu.SemaphoreType.DMA,
        ],
    )
    def gather_kernel(table_hbm, idx_hbm, out_hbm, idx_v, rows_v, sem):
        wid = lax.axis_index("s") * info.num_cores + lax.axis_index("c")
        base = wid * n_per_w
        pltpu.sync_copy(idx_hbm.at[pl.ds(base, n_per_w)], idx_v)
        pltpu.async_copy(table_hbm.at[idx_v], rows_v, sem).wait()
        pltpu.sync_copy(rows_v, out_hbm.at[pl.ds(base, n_per_w)])

    return gather_kernel(table, idx)


def _mm_body(a_ref, w_ref, b_ref, o_ref):
    acc = (
        jnp.dot(
            a_ref[...].astype(jnp.bfloat16),
            w_ref[...].astype(jnp.bfloat16),
            preferred_element_type=jnp.float32,
        )
        + b_ref[...]
    )
    o_ref[...] = acc.astype(o_ref.dtype)


def _matmul_bias(a, w_t, bias, block_m=2048, block_n=512,
                 out_dtype=jnp.float32, gate_perm=False):
    m, k = a.shape
    n = w_t.shape[1]

    if gate_perm:
        assert n // block_n == 8

        def wmap(i, j):
            return (0, 2 * (j % 4) + j // 4)
    else:
        def wmap(i, j):
            return (0, j)

    return pl.pallas_call(
        _mm_body,
        grid=(m // block_m, n // block_n),
        in_specs=[
            pl.BlockSpec((block_m, k), lambda i, j: (i, 0)),
            pl.BlockSpec((k, block_n), wmap),
            pl.BlockSpec((1, block_n), lambda i, j: (0, wmap(i, j)[1])),
        ],
        out_specs=pl.BlockSpec((block_m, block_n), lambda i, j: (i, j)),
        out_shape=jax.ShapeDtypeStruct((m, n), out_dtype),
    )(a, w_t, bias)


def _gate_src_block(j):
    return 2 * (j % 4) + j // 4


def _lstm_scan(xg, w_hh_t, unroll=8):
    s, b, g4 = xg.shape
    h_dim = w_hh_t.shape[0]
    hh = h_dim // 2
    g2 = g4 // 2
    nb = g4 // hh

    def act_half(gates, c_half):
        gi = jax.nn.sigmoid(gates[:, :hh])
        gf = jax.nn.sigmoid(gates[:, hh : 2 * hh])
        gg = jnp.tanh(gates[:, 2 * hh : 3 * hh])
        go = jax.nn.sigmoid(gates[:, 3 * hh :])
        c_new = gf * c_half + gi * gg
        h_new = go * jnp.tanh(c_new)
        return h_new, c_new

    def body(x_ref, w_ref, y_ref, h_ref, c_ref, wb_ref, ha_ref, hb_ref):
        t = pl.program_id(0)

        @pl.when(t == 0)
        def _():
            for j in range(nb):
                src = _gate_src_block(j) * hh
                wb_ref[:, j * hh : (j + 1) * hh] = (
                    w_ref[:, src : src + hh].astype(jnp.bfloat16)
                )
            h_ref[...] = jnp.zeros_like(h_ref)
            c_ref[...] = jnp.zeros_like(c_ref)
            ha_ref[...] = jnp.zeros_like(ha_ref)
            hb_ref[...] = jnp.zeros_like(hb_ref)

        h_a = ha_ref[...]
        h_b = hb_ref[...]
        c = c_ref[...]
        c_a = c[:, :hh]
        c_b = c[:, hh:]
        w = wb_ref[...]
        for u in range(unroll):
            ga = (
                x_ref[u, :, :g2]
                + jnp.dot(h_a, w[:hh, :g2], preferred_element_type=jnp.float32)
                + jnp.dot(h_b, w[hh:, :g2], preferred_element_type=jnp.float32)
            )
            gb = (
                x_ref[u, :, g2:]
                + jnp.dot(h_a, w[:hh, g2:], preferred_element_type=jnp.float32)
                + jnp.dot(h_b, w[hh:, g2:], preferred_element_type=jnp.float32)
            )
            ha_new, c_a = act_half(ga, c_a)
            h_a = ha_new.astype(jnp.bfloat16)
            hb_new, c_b = act_half(gb, c_b)
            h_b = hb_new.astype(jnp.bfloat16)
            y_ref[u, :, :hh] = h_a
            y_ref[u, :, hh:] = h_b
        h_ref[:, :hh] = ha_new
        h_ref[:, hh:] = hb_new
        c_ref[:, :hh] = c_a
        c_ref[:, hh:] = c_b
        ha_ref[...] = h_a
        hb_ref[...] = h_b

    return pl.pallas_call(
        body,
        grid=(s // unroll,),
        in_specs=[
            pl.BlockSpec((unroll, b, g4), lambda t: (t, 0, 0)),
            pl.BlockSpec((h_dim, g4), lambda t: (0, 0)),
        ],
        out_specs=[
            pl.BlockSpec((unroll, b, h_dim), lambda t: (t, 0, 0)),
            pl.BlockSpec((b, h_dim), lambda t: (0, 0)),
            pl.BlockSpec((b, h_dim), lambda t: (0, 0)),
        ],
        out_shape=[
            jax.ShapeDtypeStruct((s, b, h_dim), jnp.bfloat16),
            jax.ShapeDtypeStruct((b, h_dim), jnp.float32),
            jax.ShapeDtypeStruct((b, h_dim), jnp.float32),
        ],
        scratch_shapes=[
            pltpu.VMEM((h_dim, g4), jnp.bfloat16),
            pltpu.VMEM((b, hh), jnp.bfloat16),
            pltpu.VMEM((b, hh), jnp.bfloat16),
        ],
    )(xg, w_hh_t)


def kernel(x, emb, W_ih0, W_hh0, b_ih0, b_hh0, W_ih1, W_hh1, b_ih1, b_hh1, W_out, b_out):
    b, s = x.shape
    h_dim = W_hh0.shape[1]

    idx = x.T.reshape(-1).astype(jnp.int32)
    e = _sc_gather(emb, idx)

    xg0 = _matmul_bias(
        e, W_ih0.T, (b_ih0 + b_hh0)[None, :],
        block_m=b * s, out_dtype=jnp.bfloat16, gate_perm=True,
    )
    y0, h0, c0 = _lstm_scan(xg0.reshape(s, b, -1), W_hh0.T)

    xg1 = _matmul_bias(
        y0.reshape(s * b, h_dim), W_ih1.T, (b_ih1 + b_hh1)[None, :],
        block_m=b * s, out_dtype=jnp.bfloat16, gate_perm=True,
    )
    y1, h1, c1 = _lstm_scan(xg1.reshape(s, b, -1), W_hh1.T)

    a = jnp.transpose(y1, (1, 0, 2)).reshape(b * s, h_dim)
    out = _matmul_bias(a, W_out.T, b_out[None, :], block_m=b * s).reshape(b, s, -1)

    h_n = jnp.stack([h0, h1], axis=0)
    c_n = jnp.stack([c0, c1], axis=0)
    return (out, h_n, c_n)
